# baseline (speedup 1.0000x reference)
import functools

import jax
import jax.numpy as jnp
from jax import lax
from jax.experimental import pallas as pl
from jax.experimental.pallas import tpu as pltpu
from jax.experimental.pallas import tpu_sc as plsc

E = 160000
T = 800000
H = 128
NR = 6
SB = 7 * 6
BE = 8
IE = 64


EBLK = 6400
TBLK = 6400


def _relu(v):
    return jnp.maximum(v, 0.0)


def _dot(a, b):
    return jnp.dot(a, b, preferred_element_type=jnp.float32)


def _dot_t(at, b):
    return lax.dot_general(at, b, dimension_numbers=(((0,), (0,)), ((), ())),
                           preferred_element_type=jnp.float32)


def _pre_body(x_r, rbft_r, wr1_r, wr2_r, wkj_r, bkj_r, wji_r, bji_r, wdn_r,
              xji_o, xdn_o):
    x = x_r[...]
    rbf_t = _dot(_dot_t(rbft_r[...], wr1_r[...]), wr2_r[...])
    xji_o[...] = _relu(_dot(x, wji_r[...]) + bji_r[...])
    xkj = _relu(_dot(x, wkj_r[...]) + bkj_r[...]) * rbf_t
    xdn_o[...] = _relu(_dot(xkj, wdn_r[...]))


def _sbf_body(sbft_r, w1_r, w2_r, out_r):
    st = sbft_r[...]
    hb = st.shape[1] // 2
    w1 = w1_r[...]
    w2 = w2_r[...]
    out_r[:, 0:IE] = _dot(_dot_t(st[:, 0:hb], w1), w2)
    out_r[:, IE:2 * IE] = _dot(_dot_t(st[:, hb:2 * hb], w1), w2)


def _post_body(agg_r, xji_r, x_r, wup_r, wb1a_r, bb1a_r, wb1b_r, bb1b_r,
               wlin_r, blin_r, wa1a_r, ba1a_r, wa1b_r, ba1b_r,
               wa2a_r, ba2a_r, wa2b_r, ba2b_r, h_o):
    up = _relu(_dot(agg_r[...], wup_r[...]))
    h = xji_r[...] + up
    h = h + _relu(_dot(_relu(_dot(h, wb1a_r[...]) + bb1a_r[...]), wb1b_r[...])
                  + bb1b_r[...])
    h = _relu(_dot(h, wlin_r[...]) + blin_r[...]) + x_r[...]
    h = h + _relu(_dot(_relu(_dot(h, wa1a_r[...]) + ba1a_r[...]), wa1b_r[...])
                  + ba1b_r[...])
    h = h + _relu(_dot(_relu(_dot(h, wa2a_r[...]) + ba2a_r[...]), wa2b_r[...])
                  + ba2b_r[...])
    h_o[...] = h


def _row_spec(blk, ncol):
    return pl.BlockSpec((blk, ncol), lambda i: (i, 0))


def _rep_spec(shape):
    return pl.BlockSpec(shape, lambda i: tuple(0 for _ in shape))


NC = 2
NS = 16
NW = NC * NS
TS = T // NW
UNIT = 5000
NUNIT = TS // UNIT
UVREG = 313
NPASS = 5
PC = E // NPASS
C2 = PC // NC
ROWS_PER_TILE = C2 // NS
ZCH = 200
CH = 128
CCAP = 5376
TRASH = CCAP - 1


def _sc_body(xd_hbm, st_hbm, kj_hbm, ji_hbm, z_hbm, out_hbm,
             ji_buf, kj_buf, kj_c, t_c, dst_c, dst2d,
             xrows_a, srows_a, xrows_b, srows_b, acc,
             sem_xa, sem_sa, sem_xb, sem_sb):
    c = lax.axis_index("c")
    s = lax.axis_index("s")
    wid = c * NS + s
    iota16 = lax.iota(jnp.int32, 16)

    def pass_body(p, _):
        base_c = p * PC + c * C2

        pltpu.sync_copy(z_hbm,
                        acc.at[pl.ds(s * ROWS_PER_TILE, ROWS_PER_TILE)])
        plsc.subcore_barrier()

        def unit_body(u, _):
            toff = wid * TS + u * UNIT
            pltpu.sync_copy(ji_hbm.at[pl.ds(toff, UNIT)],
                            ji_buf.at[pl.ds(0, UNIT)])
            pltpu.sync_copy(kj_hbm.at[pl.ds(toff, UNIT)],
                            kj_buf.at[pl.ds(0, UNIT)])
            tail = ji_buf[pl.ds(UNIT - 8, 16)]
            ji_buf[pl.ds(UNIT - 8, 16)] = jnp.where(iota16 < 8, tail, -1)

            def comp_body(i, cnt):
                off = i * 16
                ji_v = ji_buf[pl.ds(off, 16)]
                kj_v = kj_buf[pl.ds(off, 16)]
                t_raw = toff + off + iota16
                b_v = t_raw // TBLK
                r_v = t_raw - b_v * TBLK
                t_v = b_v * TBLK + jnp.where(
                    r_v < TBLK // 2, 2 * r_v, 2 * (r_v - TBLK // 2) + 1)
                m = (ji_v >= base_c) & (ji_v < base_c + C2)
                mi = m.astype(jnp.int32)
                pos = plsc.cumsum(mi)
                tgt = jnp.where(m, cnt + pos - 1, TRASH)
                plsc.store_scatter(kj_c, [tgt], kj_v)
                plsc.store_scatter(t_c, [tgt], t_v)
                plsc.store_scatter(dst_c, [tgt], ji_v - base_c)
                return cnt + jnp.sum(mi)

            cnt = lax.fori_loop(0, UVREG, comp_body, jnp.int32(0))

            for k8 in range(16):
                off = cnt + k8 * 16
                kj_c[pl.ds(off, 16)] = jnp.zeros((16,), jnp.int32)
                t_c[pl.ds(off, 16)] = jnp.zeros((16,), jnp.int32)
                dst_c[pl.ds(off, 16)] = jnp.full((16,), C2, jnp.int32)
            n2 = (cnt + 2 * CH - 1) // (2 * CH)

            def cpy_body(j, _):
                for k8 in range(8):
                    v = dst_c[pl.ds(j * 128 + k8 * 16, 16)]
                    dst2d[j, pl.ds(k8 * 16, 16)] = v
                return 0
            lax.fori_loop(0, 2 * n2, cpy_body, 0)

            def issue(ch, xr, sr, semx, sems):
                pltpu.async_copy(
                    xd_hbm.at[kj_c.at[pl.ds(ch * CH, CH)]], xr, semx)
                pltpu.async_copy(
                    st_hbm.at[t_c.at[pl.ds(ch * CH, CH)]], sr, sems)

            def wait(xr, sr, semx, sems):
                pltpu.make_async_copy(xd_hbm.at[pl.ds(0, CH)], xr, semx).wait()
                pltpu.make_async_copy(st_hbm.at[pl.ds(0, CH)], sr, sems).wait()

            def mul_scatter(ch, xr, sr):
                def mul_body(r, _):
                    for f in range(4):
                        xr[r, pl.ds(f * 16, 16)] = (
                            xr[r, pl.ds(f * 16, 16)]
                            * sr[r, pl.ds(f * 16, 16)])
                    return 0
                lax.fori_loop(0, CH, mul_body, 0)
                pltpu.sync_copy(xr, acc.at[dst2d.at[ch]], add=True)

            @pl.when(n2 > 0)
            def _():
                issue(0, xrows_a, srows_a, sem_xa, sem_sa)
                issue(1, xrows_b, srows_b, sem_xb, sem_sb)

            def pair_body(g, _):
                wait(xrows_a, srows_a, sem_xa, sem_sa)
                mul_scatter(2 * g, xrows_a, srows_a)
                issue(2 * g + 2, xrows_a, srows_a, sem_xa, sem_sa)
                wait(xrows_b, srows_b, sem_xb, sem_sb)
                mul_scatter(2 * g + 1, xrows_b, srows_b)
                issue(2 * g + 3, xrows_b, srows_b, sem_xb, sem_sb)
                return 0

            lax.fori_loop(0, n2 - 1, pair_body, 0)

            @pl.when(n2 > 0)
            def _():
                wait(xrows_a, srows_a, sem_xa, sem_sa)
                mul_scatter(2 * n2 - 2, xrows_a, srows_a)
                wait(xrows_b, srows_b, sem_xb, sem_sb)
                mul_scatter(2 * n2 - 1, xrows_b, srows_b)
            return 0

        lax.fori_loop(0, NUNIT, unit_body, 0)
        plsc.subcore_barrier()

        pltpu.sync_copy(acc.at[pl.ds(s * ROWS_PER_TILE, ROWS_PER_TILE)],
                        out_hbm.at[pl.ds(base_c + s * ROWS_PER_TILE,
                                         ROWS_PER_TILE)])
        return 0

    lax.fori_loop(0, NPASS, pass_body, 0)


def _sc_scatter(x_down, sbf_t, idx_kj, idx_ji):
    mesh = plsc.VectorSubcoreMesh(core_axis_name="c", subcore_axis_name="s")
    f = pl.kernel(
        _sc_body,
        out_type=jax.ShapeDtypeStruct((E, IE), jnp.float32),
        mesh=mesh,
        compiler_params=pltpu.CompilerParams(
            needs_layout_passes=False, use_tc_tiling_on_sc=False),
        scratch_types=[
            pltpu.VMEM((UNIT + 8,), jnp.int32),
            pltpu.VMEM((UNIT + 8,), jnp.int32),
            pltpu.VMEM((CCAP,), jnp.int32),
            pltpu.VMEM((CCAP,), jnp.int32),
            pltpu.VMEM((CCAP,), jnp.int32),
            pltpu.VMEM((CCAP // CH, CH), jnp.int32),
            pltpu.VMEM((CH, IE), jnp.float32),
            pltpu.VMEM((CH, IE), jnp.float32),
            pltpu.VMEM((CH, IE), jnp.float32),
            pltpu.VMEM((CH, IE), jnp.float32),
            pltpu.VMEM_SHARED((C2 + 8, IE), jnp.float32),
            pltpu.SemaphoreType.DMA,
            pltpu.SemaphoreType.DMA,
            pltpu.SemaphoreType.DMA,
            pltpu.SemaphoreType.DMA,
        ],   )
    zeros = jnp.zeros((ROWS_PER_TILE, IE), jnp.float32)
    return f(x_down, sbf_t, idx_kj, idx_ji, zeros)


def kernel(x, rbf, sbf, W_rbf1, W_rbf2, W_sbf1, W_sbf2, W_kj, b_kj, W_ji, b_ji,
           W_down, W_up, Wb1a, bb1a, Wb1b, bb1b, W_lin, b_lin,
           Wa1a, ba1a, Wa1b, ba1b, Wa2a, ba2a, Wa2b, ba2b, idx_kj, idx_ji):
    b_kj2 = b_kj.reshape(1, H)
    b_ji2 = b_ji.reshape(1, H)
    bb1a2 = bb1a.reshape(1, H)
    bb1b2 = bb1b.reshape(1, H)
    b_lin2 = b_lin.reshape(1, H)
    ba1a2 = ba1a.reshape(1, H)
    ba1b2 = ba1b.reshape(1, H)
    ba2a2 = ba2a.reshape(1, H)
    ba2b2 = ba2b.reshape(1, H)

    n_eblk = E // EBLK
    x_ji, x_down = pl.pallas_call(
        _pre_body,
        grid=(n_eblk,),
        in_specs=[
            _row_spec(EBLK, H),
            pl.BlockSpec((NR, EBLK), lambda i: (0, i)),
            _rep_spec((NR, BE)), _rep_spec((BE, H)),
            _rep_spec((H, H)), _rep_spec((1, H)),
            _rep_spec((H, H)), _rep_spec((1, H)),
            _rep_spec((H, IE)),
        ],
        out_specs=[_row_spec(EBLK, H), _row_spec(EBLK, IE)],
        out_shape=[
            jax.ShapeDtypeStruct((E, H), jnp.float32),
            jax.ShapeDtypeStruct((E, IE), jnp.float32),
        ],
    )(x, rbf.T, W_rbf1, W_rbf2, W_kj, b_kj2, W_ji, b_ji2, W_down)

    n_tblk = T // TBLK
    sbf_t_packed = pl.pallas_call(
        _sbf_body,
        grid=(n_tblk,),
        in_specs=[
            pl.BlockSpec((SB, TBLK), lambda i: (0, i)),
            _rep_spec((SB, BE)), _rep_spec((BE, IE)),
        ],
        out_specs=_row_spec(TBLK // 2, 128),
        out_shape=jax.ShapeDtypeStruct((T // 2, 128), jnp.float32),
    )(sbf.T, W_sbf1, W_sbf2)
    sbf_t = jnp.reshape(sbf_t_packed, (T, IE))

    agg = _sc_scatter(x_down, sbf_t,
                      idx_kj.astype(jnp.int32), idx_ji.astype(jnp.int32))

    h = pl.pallas_call(
        _post_body,
        grid=(n_eblk,),
        in_specs=[
            _row_spec(EBLK, IE), _row_spec(EBLK, H), _row_spec(EBLK, H),
            _rep_spec((IE, H)),
            _rep_spec((H, H)), _rep_spec((1, H)),
            _rep_spec((H, H)), _rep_spec((1, H)),
            _rep_spec((H, H)), _rep_spec((1, H)),
            _rep_spec((H, H)), _rep_spec((1, H)),
            _rep_spec((H, H)), _rep_spec((1, H)),
            _rep_spec((H, H)), _rep_spec((1, H)),
            _rep_spec((H, H)), _rep_spec((1, H)),
        ],
        out_specs=_row_spec(EBLK, H),
        out_shape=jax.ShapeDtypeStruct((E, H), jnp.float32),
    )(agg, x_ji, x, W_up, Wb1a, bb1a2, Wb1b, bb1b2, W_lin, b_lin2,
      Wa1a, ba1a2, Wa1b, ba1b2, Wa2a, ba2a2, Wa2b, ba2b2)

    return h


# speedup vs baseline: 1.3677x; 1.3677x over previous
import functools

import jax
import jax.numpy as jnp
from jax import lax
from jax.experimental import pallas as pl
from jax.experimental.pallas import tpu as pltpu
from jax.experimental.pallas import tpu_sc as plsc

E = 160000
T = 800000
H = 128
NR = 6
SB = 7 * 6
BE = 8
IE = 64


EBLK = 6400
TBLK = 6400


def _relu(v):
    return jnp.maximum(v, 0.0)


def _dot(a, b):
    return jnp.dot(a, b, preferred_element_type=jnp.float32)


def _dot_t(at, b):
    return lax.dot_general(at, b, dimension_numbers=(((0,), (0,)), ((), ())),
                           preferred_element_type=jnp.float32)


def _pre_body(x_r, rbft_r, wr1_r, wr2_r, wkj_r, bkj_r, wji_r, bji_r, wdn_r,
              xji_o, xdn_o):
    x = x_r[...]
    rbf_t = _dot(_dot_t(rbft_r[...], wr1_r[...]), wr2_r[...])
    xji_o[...] = _relu(_dot(x, wji_r[...]) + bji_r[...])
    xkj = _relu(_dot(x, wkj_r[...]) + bkj_r[...]) * rbf_t
    xdn_o[...] = _relu(_dot(xkj, wdn_r[...]))


def _sbf_body(sbft_r, w1_r, w2_r, out_r):
    st = sbft_r[...]
    hb = st.shape[1] // 2
    w1 = w1_r[...]
    w2 = w2_r[...]
    out_r[:, 0:IE] = _dot(_dot_t(st[:, 0:hb], w1), w2)
    out_r[:, IE:2 * IE] = _dot(_dot_t(st[:, hb:2 * hb], w1), w2)


def _post_body(agg_r, xji_r, x_r, wup_r, wb1a_r, bb1a_r, wb1b_r, bb1b_r,
               wlin_r, blin_r, wa1a_r, ba1a_r, wa1b_r, ba1b_r,
               wa2a_r, ba2a_r, wa2b_r, ba2b_r, h_o):
    up = _relu(_dot(agg_r[...], wup_r[...]))
    h = xji_r[...] + up
    h = h + _relu(_dot(_relu(_dot(h, wb1a_r[...]) + bb1a_r[...]), wb1b_r[...])
                  + bb1b_r[...])
    h = _relu(_dot(h, wlin_r[...]) + blin_r[...]) + x_r[...]
    h = h + _relu(_dot(_relu(_dot(h, wa1a_r[...]) + ba1a_r[...]), wa1b_r[...])
                  + ba1b_r[...])
    h = h + _relu(_dot(_relu(_dot(h, wa2a_r[...]) + ba2a_r[...]), wa2b_r[...])
                  + ba2b_r[...])
    h_o[...] = h


def _row_spec(blk, ncol):
    return pl.BlockSpec((blk, ncol), lambda i: (i, 0))


def _rep_spec(shape):
    return pl.BlockSpec(shape, lambda i: tuple(0 for _ in shape))


NC = 2
NS = 16
NW = NC * NS
TS = T // NW
UNIT = 5000
NUNIT = TS // UNIT
UVREG = 313
NPASS = 5
PC = E // NPASS
C2 = PC // NC
ROWS_PER_TILE = C2 // NS
ZCH = 200
CH = 128
CCAP = 5248
TRASH = CCAP - 1


def _sc_body(xd_hbm, st_hbm, kj_hbm, ji_hbm, out_hbm,
             ji_buf, kj_buf, kj_c, t_c, dst_c, dst2d,
             xrows, srows, zbuf, acc, sem1, sem2):
    c = lax.axis_index("c")
    s = lax.axis_index("s")
    wid = c * NS + s
    iota16 = lax.iota(jnp.int32, 16)

    def zb_body(r, _):
        for f in range(4):
            zbuf[r, pl.ds(f * 16, 16)] = jnp.zeros((16,), jnp.float32)
        return 0
    lax.fori_loop(0, ZCH, zb_body, 0)

    def pass_body(p, _):
        base_c = p * PC + c * C2

        def z_body(z, _):
            pltpu.sync_copy(zbuf.at[pl.ds(0, ZCH)],
                            acc.at[pl.ds(s * ROWS_PER_TILE + z * ZCH, ZCH)])
            return 0
        lax.fori_loop(0, ROWS_PER_TILE // ZCH, z_body, 0)
        plsc.subcore_barrier()

        def unit_body(u, _):
            toff = wid * TS + u * UNIT
            pltpu.sync_copy(ji_hbm.at[pl.ds(toff, UNIT)],
                            ji_buf.at[pl.ds(0, UNIT)])
            pltpu.sync_copy(kj_hbm.at[pl.ds(toff, UNIT)],
                            kj_buf.at[pl.ds(0, UNIT)])
            tail = ji_buf[pl.ds(UNIT - 8, 16)]
            ji_buf[pl.ds(UNIT - 8, 16)] = jnp.where(iota16 < 8, tail, -1)

            def comp_body(i, cnt):
                off = i * 16
                ji_v = ji_buf[pl.ds(off, 16)]
                kj_v = kj_buf[pl.ds(off, 16)]
                t_raw = toff + off + iota16
                b_v = t_raw // TBLK
                r_v = t_raw - b_v * TBLK
                t_v = b_v * TBLK + jnp.where(
                    r_v < TBLK // 2, 2 * r_v, 2 * (r_v - TBLK // 2) + 1)
                m = (ji_v >= base_c) & (ji_v < base_c + C2)
                mi = m.astype(jnp.int32)
                pos = plsc.cumsum(mi)
                tgt = jnp.where(m, cnt + pos - 1, TRASH)
                plsc.store_scatter(kj_c, [tgt], kj_v)
                plsc.store_scatter(t_c, [tgt], t_v)
                plsc.store_scatter(dst_c, [tgt], ji_v - base_c)
                return cnt + jnp.sum(mi)

            cnt = lax.fori_loop(0, UVREG, comp_body, jnp.int32(0))

            for k8 in range(8):
                off = cnt + k8 * 16
                kj_c[pl.ds(off, 16)] = jnp.zeros((16,), jnp.int32)
                t_c[pl.ds(off, 16)] = jnp.zeros((16,), jnp.int32)
                dst_c[pl.ds(off, 16)] = jnp.full((16,), C2, jnp.int32)
            n_ch = (cnt + CH - 1) // CH

            def cpy_body(j, _):
                for k8 in range(8):
                    v = dst_c[pl.ds(j * 128 + k8 * 16, 16)]
                    dst2d[j, pl.ds(k8 * 16, 16)] = v
                return 0
            lax.fori_loop(0, n_ch, cpy_body, 0)

            def ch_body(ch, _):
                cb = ch * CH
                g1 = pltpu.async_copy(
                    xd_hbm.at[kj_c.at[pl.ds(cb, CH)]], xrows, sem1)
                g2 = pltpu.async_copy(
                    st_hbm.at[t_c.at[pl.ds(cb, CH)]], srows, sem2)
                g1.wait()
                g2.wait()

                def mul_body(r, _):
                    for f in range(4):
                        xrows[r, pl.ds(f * 16, 16)] = (
                            xrows[r, pl.ds(f * 16, 16)]
                            * srows[r, pl.ds(f * 16, 16)])
                    return 0
                lax.fori_loop(0, CH, mul_body, 0)

                pltpu.sync_copy(xrows, acc.at[dst2d.at[ch]], add=True)
                return 0

            lax.fori_loop(0, n_ch, ch_body, 0)
            return 0

        lax.fori_loop(0, NUNIT, unit_body, 0)
        plsc.subcore_barrier()

        pltpu.sync_copy(acc.at[pl.ds(s * ROWS_PER_TILE, ROWS_PER_TILE)],
                        out_hbm.at[pl.ds(base_c + s * ROWS_PER_TILE,
                                         ROWS_PER_TILE)])
        return 0

    lax.fori_loop(0, NPASS, pass_body, 0)


def _sc_scatter(x_down, sbf_t, idx_kj, idx_ji):
    mesh = plsc.VectorSubcoreMesh(core_axis_name="c", subcore_axis_name="s")
    f = pl.kernel(
        _sc_body,
        out_type=jax.ShapeDtypeStruct((E, IE), jnp.float32),
        mesh=mesh,
        compiler_params=pltpu.CompilerParams(
            needs_layout_passes=False, use_tc_tiling_on_sc=False),
        scratch_types=[
            pltpu.VMEM((UNIT + 8,), jnp.int32),
            pltpu.VMEM((UNIT + 8,), jnp.int32),
            pltpu.VMEM((CCAP,), jnp.int32),
            pltpu.VMEM((CCAP,), jnp.int32),
            pltpu.VMEM((CCAP,), jnp.int32),
            pltpu.VMEM((CCAP // CH, CH), jnp.int32),
            pltpu.VMEM((CH, IE), jnp.float32),
            pltpu.VMEM((CH, IE), jnp.float32),
            pltpu.VMEM((ZCH, IE), jnp.float32),
            pltpu.VMEM_SHARED((C2 + 8, IE), jnp.float32),
            pltpu.SemaphoreType.DMA,
            pltpu.SemaphoreType.DMA,
        ],  )
    return f(x_down, sbf_t, idx_kj, idx_ji)


def kernel(x, rbf, sbf, W_rbf1, W_rbf2, W_sbf1, W_sbf2, W_kj, b_kj, W_ji, b_ji,
           W_down, W_up, Wb1a, bb1a, Wb1b, bb1b, W_lin, b_lin,
           Wa1a, ba1a, Wa1b, ba1b, Wa2a, ba2a, Wa2b, ba2b, idx_kj, idx_ji):
    b_kj2 = b_kj.reshape(1, H)
    b_ji2 = b_ji.reshape(1, H)
    bb1a2 = bb1a.reshape(1, H)
    bb1b2 = bb1b.reshape(1, H)
    b_lin2 = b_lin.reshape(1, H)
    ba1a2 = ba1a.reshape(1, H)
    ba1b2 = ba1b.reshape(1, H)
    ba2a2 = ba2a.reshape(1, H)
    ba2b2 = ba2b.reshape(1, H)

    n_eblk = E // EBLK
    x_ji, x_down = pl.pallas_call(
        _pre_body,
        grid=(n_eblk,),
        in_specs=[
            _row_spec(EBLK, H),
            pl.BlockSpec((NR, EBLK), lambda i: (0, i)),
            _rep_spec((NR, BE)), _rep_spec((BE, H)),
            _rep_spec((H, H)), _rep_spec((1, H)),
            _rep_spec((H, H)), _rep_spec((1, H)),
            _rep_spec((H, IE)),
        ],
        out_specs=[_row_spec(EBLK, H), _row_spec(EBLK, IE)],
        out_shape=[
            jax.ShapeDtypeStruct((E, H), jnp.float32),
            jax.ShapeDtypeStruct((E, IE), jnp.float32),
        ],
    )(x, rbf.T, W_rbf1, W_rbf2, W_kj, b_kj2, W_ji, b_ji2, W_down)

    n_tblk = T // TBLK
    sbf_t_packed = pl.pallas_call(
        _sbf_body,
        grid=(n_tblk,),
        in_specs=[
            pl.BlockSpec((SB, TBLK), lambda i: (0, i)),
            _rep_spec((SB, BE)), _rep_spec((BE, IE)),
        ],
        out_specs=_row_spec(TBLK // 2, 128),
        out_shape=jax.ShapeDtypeStruct((T // 2, 128), jnp.float32),
    )(sbf.T, W_sbf1, W_sbf2)
    sbf_t = jnp.reshape(sbf_t_packed, (T, IE))

    agg = _sc_scatter(x_down, sbf_t,
                      idx_kj.astype(jnp.int32), idx_ji.astype(jnp.int32))

    h = pl.pallas_call(
        _post_body,
        grid=(n_eblk,),
        in_specs=[
            _row_spec(EBLK, IE), _row_spec(EBLK, H), _row_spec(EBLK, H),
            _rep_spec((IE, H)),
            _rep_spec((H, H)), _rep_spec((1, H)),
            _rep_spec((H, H)), _rep_spec((1, H)),
            _rep_spec((H, H)), _rep_spec((1, H)),
            _rep_spec((H, H)), _rep_spec((1, H)),
            _rep_spec((H, H)), _rep_spec((1, H)),
            _rep_spec((H, H)), _rep_spec((1, H)),
            _rep_spec((H, H)), _rep_spec((1, H)),
        ],
        out_specs=_row_spec(EBLK, H),
        out_shape=jax.ShapeDtypeStruct((E, H), jnp.float32),
    )(agg, x_ji, x, W_up, Wb1a, bb1a2, Wb1b, bb1b2, W_lin, b_lin2,
      Wa1a, ba1a2, Wa1b, ba1b2, Wa2a, ba2a2, Wa2b, ba2b2)

    return h


# speedup vs baseline: 1.3686x; 1.0007x over previous
import functools

import jax
import jax.numpy as jnp
from jax import lax
from jax.experimental import pallas as pl
from jax.experimental.pallas import tpu as pltpu
from jax.experimental.pallas import tpu_sc as plsc

E = 160000
T = 800000
H = 128
NR = 6
SB = 7 * 6
BE = 8
IE = 64


EBLK = 6400
TBLK = 6400


def _relu(v):
    return jnp.maximum(v, 0.0)


def _dot(a, b):
    return jnp.dot(a, b, preferred_element_type=jnp.float32)


def _dot_t(at, b):
    return lax.dot_general(at, b, dimension_numbers=(((0,), (0,)), ((), ())),
                           preferred_element_type=jnp.float32)


def _pre_body(x_r, rbft_r, wr1_r, wr2_r, wkj_r, bkj_r, wji_r, bji_r, wdn_r,
              xji_o, xdn_o):
    x = x_r[...]
    rbf_t = _dot(_dot_t(rbft_r[...], wr1_r[...]), wr2_r[...])
    xji_o[...] = _relu(_dot(x, wji_r[...]) + bji_r[...])
    xkj = _relu(_dot(x, wkj_r[...]) + bkj_r[...]) * rbf_t
    xdn_o[...] = _relu(_dot(xkj, wdn_r[...]))


def _sbf_body(sbft_r, w1_r, w2_r, out_r):
    st = sbft_r[...]
    hb = st.shape[1] // 2
    w1 = w1_r[...]
    w2 = w2_r[...]
    out_r[:, 0:IE] = _dot(_dot_t(st[:, 0:hb], w1), w2)
    out_r[:, IE:2 * IE] = _dot(_dot_t(st[:, hb:2 * hb], w1), w2)


def _post_body(agg_r, xji_r, x_r, wup_r, wb1a_r, bb1a_r, wb1b_r, bb1b_r,
               wlin_r, blin_r, wa1a_r, ba1a_r, wa1b_r, ba1b_r,
               wa2a_r, ba2a_r, wa2b_r, ba2b_r, h_o):
    up = _relu(_dot(agg_r[...], wup_r[...]))
    h = xji_r[...] + up
    h = h + _relu(_dot(_relu(_dot(h, wb1a_r[...]) + bb1a_r[...]), wb1b_r[...])
                  + bb1b_r[...])
    h = _relu(_dot(h, wlin_r[...]) + blin_r[...]) + x_r[...]
    h = h + _relu(_dot(_relu(_dot(h, wa1a_r[...]) + ba1a_r[...]), wa1b_r[...])
                  + ba1b_r[...])
    h = h + _relu(_dot(_relu(_dot(h, wa2a_r[...]) + ba2a_r[...]), wa2b_r[...])
                  + ba2b_r[...])
    h_o[...] = h


def _row_spec(blk, ncol):
    return pl.BlockSpec((blk, ncol), lambda i: (i, 0))


def _rep_spec(shape):
    return pl.BlockSpec(shape, lambda i: tuple(0 for _ in shape))


NC = 2
NS = 16
NW = NC * NS
TS = T // NW
UNIT = 5000
NUNIT = TS // UNIT
UVREG = 313
NPASS = 5
PC = E // NPASS
C2 = PC // NC
ROWS_PER_TILE = C2 // NS
ZCH = 200
CH = 128
CCAP = 5248
TRASH = CCAP - 1


def _sc_body(xd_hbm, st_hbm, kj_hbm, ji_hbm, out_hbm,
             ji_buf, kj_buf, kj_c, t_c, dst_c, dst2d,
             xrows, srows, unused_a, unused_b, zbuf, acc, sem1, sem2,
             sem3, sem4):
    c = lax.axis_index("c")
    s = lax.axis_index("s")
    wid = c * NS + s
    iota16 = lax.iota(jnp.int32, 16)

    def zb_body(r, _):
        for f in range(4):
            zbuf[r, pl.ds(f * 16, 16)] = jnp.zeros((16,), jnp.float32)
        return 0
    lax.fori_loop(0, ZCH, zb_body, 0)

    def pass_body(p, _):
        base_c = p * PC + c * C2

        def z_body(z, _):
            pltpu.sync_copy(zbuf.at[pl.ds(0, ZCH)],
                            acc.at[pl.ds(s * ROWS_PER_TILE + z * ZCH, ZCH)])
            return 0
        lax.fori_loop(0, ROWS_PER_TILE // ZCH, z_body, 0)
        plsc.subcore_barrier()

        def unit_body(u, _):
            toff = wid * TS + u * UNIT
            pltpu.sync_copy(ji_hbm.at[pl.ds(toff, UNIT)],
                            ji_buf.at[pl.ds(0, UNIT)])
            pltpu.sync_copy(kj_hbm.at[pl.ds(toff, UNIT)],
                            kj_buf.at[pl.ds(0, UNIT)])
            tail = ji_buf[pl.ds(UNIT - 8, 16)]
            ji_buf[pl.ds(UNIT - 8, 16)] = jnp.where(iota16 < 8, tail, -1)

            def comp_body(i, cnt):
                off = i * 16
                ji_v = ji_buf[pl.ds(off, 16)]
                kj_v = kj_buf[pl.ds(off, 16)]
                t_raw = toff + off + iota16
                b_v = t_raw // TBLK
                r_v = t_raw - b_v * TBLK
                t_v = b_v * TBLK + jnp.where(
                    r_v < TBLK // 2, 2 * r_v, 2 * (r_v - TBLK // 2) + 1)
                m = (ji_v >= base_c) & (ji_v < base_c + C2)
                mi = m.astype(jnp.int32)
                pos = plsc.cumsum(mi)
                tgt = jnp.where(m, cnt + pos - 1, TRASH)
                plsc.store_scatter(kj_c, [tgt], kj_v)
                plsc.store_scatter(t_c, [tgt], t_v)
                plsc.store_scatter(dst_c, [tgt], ji_v - base_c)
                return cnt + jnp.sum(mi)

            cnt = lax.fori_loop(0, UVREG, comp_body, jnp.int32(0))

            for k8 in range(8):
                off = cnt + k8 * 16
                kj_c[pl.ds(off, 16)] = jnp.zeros((16,), jnp.int32)
                t_c[pl.ds(off, 16)] = jnp.zeros((16,), jnp.int32)
                dst_c[pl.ds(off, 16)] = jnp.full((16,), C2, jnp.int32)
            n_ch = (cnt + CH - 1) // CH

            def cpy_body(j, _):
                for k8 in range(8):
                    v = dst_c[pl.ds(j * 128 + k8 * 16, 16)]
                    dst2d[j, pl.ds(k8 * 16, 16)] = v
                return 0
            lax.fori_loop(0, n_ch, cpy_body, 0)

            def ch_body(ch, _):
                cb = ch * CH
                g1 = pltpu.async_copy(
                    xd_hbm.at[kj_c.at[pl.ds(cb, CH)]], xrows, sem1)
                g2 = pltpu.async_copy(
                    st_hbm.at[t_c.at[pl.ds(cb, CH)]], srows, sem2)
                g1.wait()
                g2.wait()

                def mul_body(r, _):
                    for f in range(4):
                        xrows[r, pl.ds(f * 16, 16)] = (
                            xrows[r, pl.ds(f * 16, 16)]
                            * srows[r, pl.ds(f * 16, 16)])
                    return 0
                lax.fori_loop(0, CH, mul_body, 0)

                pltpu.sync_copy(xrows, acc.at[dst2d.at[ch]], add=True)
                return 0

            lax.fori_loop(0, n_ch, ch_body, 0)
            return 0

        lax.fori_loop(0, NUNIT, unit_body, 0)
        plsc.subcore_barrier()

        pltpu.sync_copy(acc.at[pl.ds(s * ROWS_PER_TILE, ROWS_PER_TILE)],
                        out_hbm.at[pl.ds(base_c + s * ROWS_PER_TILE,
                                         ROWS_PER_TILE)])
        return 0

    lax.fori_loop(0, NPASS, pass_body, 0)


def _sc_scatter(x_down, sbf_t, idx_kj, idx_ji):
    mesh = plsc.VectorSubcoreMesh(core_axis_name="c", subcore_axis_name="s")
    f = pl.kernel(
        _sc_body,
        out_type=jax.ShapeDtypeStruct((E, IE), jnp.float32),
        mesh=mesh,
        compiler_params=pltpu.CompilerParams(
            needs_layout_passes=False, use_tc_tiling_on_sc=False),
        scratch_types=[
            pltpu.VMEM((UNIT + 8,), jnp.int32),
            pltpu.VMEM((UNIT + 8,), jnp.int32),
            pltpu.VMEM((CCAP,), jnp.int32),
            pltpu.VMEM((CCAP,), jnp.int32),
            pltpu.VMEM((CCAP,), jnp.int32),
            pltpu.VMEM((CCAP // CH, CH), jnp.int32),
            pltpu.VMEM((CH, IE), jnp.float32),
            pltpu.VMEM((CH, IE), jnp.float32),
            pltpu.VMEM((CH, IE), jnp.float32),
            pltpu.VMEM((CH, IE), jnp.float32),
            pltpu.VMEM((ZCH, IE), jnp.float32),
            pltpu.VMEM_SHARED((C2 + 8, IE), jnp.float32),
            pltpu.SemaphoreType.DMA,
            pltpu.SemaphoreType.DMA,
            pltpu.SemaphoreType.DMA,
            pltpu.SemaphoreType.DMA,
        ],  )
    return f(x_down, sbf_t, idx_kj, idx_ji)


def kernel(x, rbf, sbf, W_rbf1, W_rbf2, W_sbf1, W_sbf2, W_kj, b_kj, W_ji, b_ji,
           W_down, W_up, Wb1a, bb1a, Wb1b, bb1b, W_lin, b_lin,
           Wa1a, ba1a, Wa1b, ba1b, Wa2a, ba2a, Wa2b, ba2b, idx_kj, idx_ji):
    b_kj2 = b_kj.reshape(1, H)
    b_ji2 = b_ji.reshape(1, H)
    bb1a2 = bb1a.reshape(1, H)
    bb1b2 = bb1b.reshape(1, H)
    b_lin2 = b_lin.reshape(1, H)
    ba1a2 = ba1a.reshape(1, H)
    ba1b2 = ba1b.reshape(1, H)
    ba2a2 = ba2a.reshape(1, H)
    ba2b2 = ba2b.reshape(1, H)

    n_eblk = E // EBLK
    x_ji, x_down = pl.pallas_call(
        _pre_body,
        grid=(n_eblk,),
        in_specs=[
            _row_spec(EBLK, H),
            pl.BlockSpec((NR, EBLK), lambda i: (0, i)),
            _rep_spec((NR, BE)), _rep_spec((BE, H)),
            _rep_spec((H, H)), _rep_spec((1, H)),
            _rep_spec((H, H)), _rep_spec((1, H)),
            _rep_spec((H, IE)),
        ],
        out_specs=[_row_spec(EBLK, H), _row_spec(EBLK, IE)],
        out_shape=[
            jax.ShapeDtypeStruct((E, H), jnp.float32),
            jax.ShapeDtypeStruct((E, IE), jnp.float32),
        ],
    )(x, rbf.T, W_rbf1, W_rbf2, W_kj, b_kj2, W_ji, b_ji2, W_down)

    n_tblk = T // TBLK
    sbf_t_packed = pl.pallas_call(
        _sbf_body,
        grid=(n_tblk,),
        in_specs=[
            pl.BlockSpec((SB, TBLK), lambda i: (0, i)),
            _rep_spec((SB, BE)), _rep_spec((BE, IE)),
        ],
        out_specs=_row_spec(TBLK // 2, 128),
        out_shape=jax.ShapeDtypeStruct((T // 2, 128), jnp.float32),
    )(sbf.T, W_sbf1, W_sbf2)
    sbf_t = jnp.reshape(sbf_t_packed, (T, IE))

    agg = _sc_scatter(x_down, sbf_t,
                      idx_kj.astype(jnp.int32), idx_ji.astype(jnp.int32))

    h = pl.pallas_call(
        _post_body,
        grid=(n_eblk,),
        in_specs=[
            _row_spec(EBLK, IE), _row_spec(EBLK, H), _row_spec(EBLK, H),
            _rep_spec((IE, H)),
            _rep_spec((H, H)), _rep_spec((1, H)),
            _rep_spec((H, H)), _rep_spec((1, H)),
            _rep_spec((H, H)), _rep_spec((1, H)),
            _rep_spec((H, H)), _rep_spec((1, H)),
            _rep_spec((H, H)), _rep_spec((1, H)),
            _rep_spec((H, H)), _rep_spec((1, H)),
            _rep_spec((H, H)), _rep_spec((1, H)),
        ],
        out_specs=_row_spec(EBLK, H),
        out_shape=jax.ShapeDtypeStruct((E, H), jnp.float32),
    )(agg, x_ji, x, W_up, Wb1a, bb1a2, Wb1b, bb1b2, W_lin, b_lin2,
      Wa1a, ba1a2, Wa1b, ba1b2, Wa2a, ba2a2, Wa2b, ba2b2)

    return h
